# Initial kernel scaffold; baseline (speedup 1.0000x reference)
import jax
import jax.numpy as jnp
from jax.experimental import pallas as pl


def kernel(hidden_states, c_attn_w, c_attn_b, c_proj_w, c_proj_b):
    raise NotImplementedError("write your pallas kernel here")


# speedup vs baseline: 3.6140x; 3.6140x over previous
import jax
import jax.numpy as jnp
from jax.experimental import pallas as pl
from jax.experimental.pallas import tpu as pltpu

_NUM_HEADS = 12
_HEAD_DIM = 64
_EMBED = 768
_GROUP = 4
_CHUNK = 512
_INV_SQRT = 1.0 / (_HEAD_DIM ** 0.5)

_DN_T = (((1,), (1,)), ((), ()))


def _dequant_keys(k):
    a = jnp.abs(k)
    n = k.shape[1]
    w = a
    for r in (1, 2, 3):
        w = jnp.maximum(w, pltpu.roll(a, n - r, 1))
    lane = jax.lax.broadcasted_iota(jnp.int32, k.shape, 1)
    first = (lane % _GROUP) == 0
    m0 = jnp.where(first, w, 0.0)
    gm = m0
    for r in (1, 2, 3):
        gm = gm + pltpu.roll(m0, r, 1)
    scale = jnp.where(gm > 0.0, gm, 1.0)
    kq = jnp.clip(jnp.round(k / scale), -2.0, 1.0)
    return kq * scale


def _attn_kernel(x_ref, wq_ref, wk_ref, wv_ref, bq_ref, bk_ref, bv_ref,
                 wp_ref, bp_ref, out_ref):
    h = pl.program_id(0)

    @pl.when(h == 0)
    def _init():
        out_ref[...] = jnp.broadcast_to(bp_ref[...], out_ref.shape)

    x = x_ref[...]
    k = jax.lax.dot_general(x, wk_ref[0], _DN_T) + bk_ref[0]
    v = jax.lax.dot_general(x, wv_ref[0], _DN_T) + bv_ref[0]
    kd = _dequant_keys(k)
    seq = x.shape[0]
    for c in range(seq // _CHUNK):
        rows = pl.ds(c * _CHUNK, _CHUNK)
        xc = x[c * _CHUNK:(c + 1) * _CHUNK, :]
        q = jax.lax.dot_general(xc, wq_ref[0], _DN_T) + bq_ref[0]
        s = jax.lax.dot_general(q, kd, _DN_T) * _INV_SQRT
        s = s - jnp.max(s, axis=-1, keepdims=True)
        p = jnp.exp(s)
        p = p / jnp.sum(p, axis=-1, keepdims=True)
        o = jnp.dot(p, v)
        out_ref[rows, :] += jnp.dot(o, wp_ref[...])


def kernel(hidden_states, c_attn_w, c_attn_b, c_proj_w, c_proj_b):
    B, S, E = hidden_states.shape
    x = hidden_states.reshape(S, E)
    cw = c_attn_w.T.reshape(3 * _NUM_HEADS, _HEAD_DIM, E)
    cb = c_attn_b.reshape(3 * _NUM_HEADS, 1, _HEAD_DIM)
    pb = c_proj_b.reshape(1, E)
    out = pl.pallas_call(
        _attn_kernel,
        grid=(_NUM_HEADS,),
        in_specs=[
            pl.BlockSpec((S, E), lambda h: (0, 0)),
            pl.BlockSpec((1, _HEAD_DIM, E), lambda h: (h, 0, 0)),
            pl.BlockSpec((1, _HEAD_DIM, E), lambda h: (_NUM_HEADS + h, 0, 0)),
            pl.BlockSpec((1, _HEAD_DIM, E), lambda h: (2 * _NUM_HEADS + h, 0, 0)),
            pl.BlockSpec((1, 1, _HEAD_DIM), lambda h: (h, 0, 0)),
            pl.BlockSpec((1, 1, _HEAD_DIM), lambda h: (_NUM_HEADS + h, 0, 0)),
            pl.BlockSpec((1, 1, _HEAD_DIM), lambda h: (2 * _NUM_HEADS + h, 0, 0)),
            pl.BlockSpec((_HEAD_DIM, E), lambda h: (h, 0)),
            pl.BlockSpec((1, E), lambda h: (0, 0)),
        ],
        out_specs=pl.BlockSpec((S, E), lambda h: (0, 0)),
        out_shape=jax.ShapeDtypeStruct((S, E), jnp.float32),
        compiler_params=pltpu.CompilerParams(
            dimension_semantics=("arbitrary",),
            vmem_limit_bytes=100 * 1024 * 1024,
        ),
    )(x, cw, cw, cw, cb, cb, cb, c_proj_w, pb)
    return out.reshape(B, S, E)


# speedup vs baseline: 3.8291x; 1.0595x over previous
import jax
import jax.numpy as jnp
from jax.experimental import pallas as pl
from jax.experimental.pallas import tpu as pltpu

_NUM_HEADS = 12
_HEAD_DIM = 64
_EMBED = 768
_GROUP = 4
_CHUNK = 512
_INV_SQRT = 1.0 / (_HEAD_DIM ** 0.5)

_DN_T = (((1,), (1,)), ((), ()))


def _dequant_keys(k):
    a = jnp.abs(k)
    n = k.shape[1]
    w = a
    for r in (1, 2, 3):
        w = jnp.maximum(w, pltpu.roll(a, n - r, 1))
    lane = jax.lax.broadcasted_iota(jnp.int32, k.shape, 1)
    first = (lane % _GROUP) == 0
    m0 = jnp.where(first, w, 0.0)
    gm = m0
    for r in (1, 2, 3):
        gm = gm + pltpu.roll(m0, r, 1)
    scale = jnp.where(gm > 0.0, gm, 1.0)
    kq = jnp.clip(jnp.round(k / scale), -2.0, 1.0)
    return kq * scale


def _attn_kernel(x_ref, wq_ref, wk_ref, wv_ref, bq_ref, bk_ref, bv_ref,
                 wp_ref, bp_ref, out_ref):
    h = pl.program_id(0)

    @pl.when(h == 0)
    def _init():
        out_ref[...] = jnp.broadcast_to(bp_ref[...], out_ref.shape)

    x = x_ref[...]
    k = jax.lax.dot_general(x, wk_ref[0], _DN_T) + bk_ref[0]
    v = jax.lax.dot_general(x, wv_ref[0], _DN_T) + bv_ref[0]
    kd = _dequant_keys(k)
    seq = x.shape[0]
    for c in range(seq // _CHUNK):
        rows = pl.ds(c * _CHUNK, _CHUNK)
        xc = x_ref[rows, :]
        q = (jax.lax.dot_general(xc, wq_ref[0], _DN_T) + bq_ref[0]) * _INV_SQRT
        s = jax.lax.dot_general(q, kd, _DN_T)
        e = jnp.exp(s - jnp.max(s, axis=-1, keepdims=True))
        o = jnp.dot(e, v) / jnp.sum(e, axis=-1, keepdims=True)
        out_ref[rows, :] += jnp.dot(o, wp_ref[...])


def kernel(hidden_states, c_attn_w, c_attn_b, c_proj_w, c_proj_b):
    B, S, E = hidden_states.shape
    x = hidden_states.reshape(S, E)
    cw = c_attn_w.T.reshape(3 * _NUM_HEADS, _HEAD_DIM, E)
    cb = c_attn_b.reshape(3 * _NUM_HEADS, 1, _HEAD_DIM)
    pb = c_proj_b.reshape(1, E)
    out = pl.pallas_call(
        _attn_kernel,
        grid=(_NUM_HEADS,),
        in_specs=[
            pl.BlockSpec((S, E), lambda h: (0, 0)),
            pl.BlockSpec((1, _HEAD_DIM, E), lambda h: (h, 0, 0)),
            pl.BlockSpec((1, _HEAD_DIM, E), lambda h: (_NUM_HEADS + h, 0, 0)),
            pl.BlockSpec((1, _HEAD_DIM, E), lambda h: (2 * _NUM_HEADS + h, 0, 0)),
            pl.BlockSpec((1, 1, _HEAD_DIM), lambda h: (h, 0, 0)),
            pl.BlockSpec((1, 1, _HEAD_DIM), lambda h: (_NUM_HEADS + h, 0, 0)),
            pl.BlockSpec((1, 1, _HEAD_DIM), lambda h: (2 * _NUM_HEADS + h, 0, 0)),
            pl.BlockSpec((_HEAD_DIM, E), lambda h: (h, 0)),
            pl.BlockSpec((1, E), lambda h: (0, 0)),
        ],
        out_specs=pl.BlockSpec((S, E), lambda h: (0, 0)),
        out_shape=jax.ShapeDtypeStruct((S, E), jnp.float32),
        compiler_params=pltpu.CompilerParams(
            dimension_semantics=("arbitrary",),
            vmem_limit_bytes=100 * 1024 * 1024,
        ),
    )(x, cw, cw, cw, cb, cb, cb, c_proj_w, pb)
    return out.reshape(B, S, E)


# speedup vs baseline: 5.6051x; 1.4638x over previous
import jax
import jax.numpy as jnp
from jax.experimental import pallas as pl
from jax.experimental.pallas import tpu as pltpu

_NUM_HEADS = 12
_HEAD_DIM = 64
_EMBED = 768
_GROUP = 4
_CHUNK = 512
_HPP = 2
_W = _HPP * _HEAD_DIM
_INV_SQRT = 1.0 / (_HEAD_DIM ** 0.5)

_DN_T = (((1,), (1,)), ((), ()))


def _dequant_keys(k):
    a = jnp.abs(k)
    n = k.shape[1]
    w = a
    for r in (1, 2, 3):
        w = jnp.maximum(w, pltpu.roll(a, n - r, 1))
    lane = jax.lax.broadcasted_iota(jnp.int32, k.shape, 1)
    first = (lane % _GROUP) == 0
    m0 = jnp.where(first, w, 0.0)
    gm = m0
    for r in (1, 2, 3):
        gm = gm + pltpu.roll(m0, r, 1)
    scale = jnp.where(gm > 0.0, gm, 1.0)
    kq = jnp.clip(jnp.round(k / scale), -2.0, 1.0)
    return kq * scale


def _attn_kernel(x_ref, wq_ref, wk_ref, wv_ref, bq_ref, bk_ref, bv_ref,
                 wp_ref, bp_ref, out_ref):
    g = pl.program_id(0)

    @pl.when(g == 0)
    def _init():
        out_ref[...] = jnp.broadcast_to(bp_ref[...], out_ref.shape)

    x = x_ref[...]
    wq, wk, wv = wq_ref[0], wk_ref[0], wv_ref[0]
    bq, bk, bv = bq_ref[0], bk_ref[0], bv_ref[0]
    k2 = jax.lax.dot_general(x, wk, _DN_T) + bk
    v2 = jax.lax.dot_general(x, wv, _DN_T) + bv
    kd2 = _dequant_keys(k2)
    seq = x.shape[0]
    for c in range(seq // _CHUNK):
        rows = pl.ds(c * _CHUNK, _CHUNK)
        xc = x_ref[rows, :]
        q2 = (jax.lax.dot_general(xc, wq, _DN_T) + bq) * _INV_SQRT
        outs = []
        for hh in range(_HPP):
            lanes = slice(hh * _HEAD_DIM, (hh + 1) * _HEAD_DIM)
            s = jax.lax.dot_general(q2[:, lanes], kd2[:, lanes], _DN_T)
            e = jnp.exp(s - jnp.max(s, axis=-1, keepdims=True))
            outs.append(jnp.dot(e, v2[:, lanes])
                        / jnp.sum(e, axis=-1, keepdims=True))
        o2 = jnp.concatenate(outs, axis=1)
        out_ref[rows, :] += jnp.dot(o2, wp_ref[...])


def kernel(hidden_states, c_attn_w, c_attn_b, c_proj_w, c_proj_b):
    B, S, E = hidden_states.shape
    n_prog = _NUM_HEADS // _HPP
    x = hidden_states.reshape(S, E)
    cw = c_attn_w.T.reshape(3 * n_prog, _W, E)
    cb = c_attn_b.reshape(3 * n_prog, 1, _W)
    pb = c_proj_b.reshape(1, E)
    out = pl.pallas_call(
        _attn_kernel,
        grid=(n_prog,),
        in_specs=[
            pl.BlockSpec((S, E), lambda g: (0, 0)),
            pl.BlockSpec((1, _W, E), lambda g: (g, 0, 0)),
            pl.BlockSpec((1, _W, E), lambda g: (n_prog + g, 0, 0)),
            pl.BlockSpec((1, _W, E), lambda g: (2 * n_prog + g, 0, 0)),
            pl.BlockSpec((1, 1, _W), lambda g: (g, 0, 0)),
            pl.BlockSpec((1, 1, _W), lambda g: (n_prog + g, 0, 0)),
            pl.BlockSpec((1, 1, _W), lambda g: (2 * n_prog + g, 0, 0)),
            pl.BlockSpec((_W, E), lambda g: (g, 0)),
            pl.BlockSpec((1, E), lambda g: (0, 0)),
        ],
        out_specs=pl.BlockSpec((S, E), lambda g: (0, 0)),
        out_shape=jax.ShapeDtypeStruct((S, E), jnp.float32),
        compiler_params=pltpu.CompilerParams(
            dimension_semantics=("arbitrary",),
            vmem_limit_bytes=100 * 1024 * 1024,
        ),
    )(x, cw, cw, cw, cb, cb, cb, c_proj_w, pb)
    return out.reshape(B, S, E)


# speedup vs baseline: 5.6309x; 1.0046x over previous
import jax
import jax.numpy as jnp
from jax.experimental import pallas as pl
from jax.experimental.pallas import tpu as pltpu

_NUM_HEADS = 12
_HEAD_DIM = 64
_EMBED = 768
_GROUP = 4
_CHUNK = 512
_HPP = 2
_W = _HPP * _HEAD_DIM
_INV_SQRT = 1.0 / (_HEAD_DIM ** 0.5)

_DN_T = (((1,), (1,)), ((), ()))


def _dequant_keys(k):
    a = jnp.abs(k)
    n = k.shape[1]
    w = a
    for r in (1, 2, 3):
        w = jnp.maximum(w, pltpu.roll(a, n - r, 1))
    lane = jax.lax.broadcasted_iota(jnp.int32, k.shape, 1)
    first = (lane % _GROUP) == 0
    m0 = jnp.where(first, w, 0.0)
    gm = m0
    for r in (1, 2, 3):
        gm = gm + pltpu.roll(m0, r, 1)
    scale = jnp.where(gm > 0.0, gm, 1.0)
    kq = jnp.clip(jnp.round(k / scale), -2.0, 1.0)
    return kq * scale


def _attn_kernel(x_ref, wq_ref, wk_ref, wv_ref, bq_ref, bk_ref, bv_ref,
                 wp_ref, bp_ref, out_ref):
    g = pl.program_id(0)

    @pl.when(g == 0)
    def _init():
        out_ref[...] = jnp.broadcast_to(bp_ref[...], out_ref.shape)

    x = x_ref[...]
    wq, wk, wv = wq_ref[0], wk_ref[0], wv_ref[0]
    bq, bk, bv = bq_ref[0], bk_ref[0], bv_ref[0]
    xb = x.astype(jnp.bfloat16)
    k2 = jax.lax.dot_general(x, wk, _DN_T) + bk
    v2b = (jax.lax.dot_general(xb, wv.astype(jnp.bfloat16), _DN_T,
                               preferred_element_type=jnp.float32)
           + bv).astype(jnp.bfloat16)
    kd2b = _dequant_keys(k2).astype(jnp.bfloat16)
    wqb = wq.astype(jnp.bfloat16)
    wpb = wp_ref[...].astype(jnp.bfloat16)
    seq = x.shape[0]
    for c in range(seq // _CHUNK):
        rows = pl.ds(c * _CHUNK, _CHUNK)
        xcb = x_ref[rows, :].astype(jnp.bfloat16)
        q2 = (jax.lax.dot_general(xcb, wqb, _DN_T,
                                  preferred_element_type=jnp.float32)
              + bq) * _INV_SQRT
        q2b = q2.astype(jnp.bfloat16)
        outs = []
        for hh in range(_HPP):
            lanes = slice(hh * _HEAD_DIM, (hh + 1) * _HEAD_DIM)
            s = jax.lax.dot_general(q2b[:, lanes], kd2b[:, lanes], _DN_T,
                                    preferred_element_type=jnp.float32)
            e = jnp.exp(s - jnp.max(s, axis=-1, keepdims=True))
            eb = e.astype(jnp.bfloat16)
            outs.append(jax.lax.dot_general(
                eb, v2b, (((1,), (0,)), ((), ())),
                preferred_element_type=jnp.float32)[:, lanes]
                / jnp.sum(e, axis=-1, keepdims=True))
        o2 = jnp.concatenate(outs, axis=1).astype(jnp.bfloat16)
        out_ref[rows, :] += jax.lax.dot_general(
            o2, wpb, (((1,), (0,)), ((), ())),
            preferred_element_type=jnp.float32)


def kernel(hidden_states, c_attn_w, c_attn_b, c_proj_w, c_proj_b):
    B, S, E = hidden_states.shape
    n_prog = _NUM_HEADS // _HPP
    x = hidden_states.reshape(S, E)
    cw = c_attn_w.T.reshape(3 * n_prog, _W, E)
    cb = c_attn_b.reshape(3 * n_prog, 1, _W)
    pb = c_proj_b.reshape(1, E)
    out = pl.pallas_call(
        _attn_kernel,
        grid=(n_prog,),
        in_specs=[
            pl.BlockSpec((S, E), lambda g: (0, 0)),
            pl.BlockSpec((1, _W, E), lambda g: (g, 0, 0)),
            pl.BlockSpec((1, _W, E), lambda g: (n_prog + g, 0, 0)),
            pl.BlockSpec((1, _W, E), lambda g: (2 * n_prog + g, 0, 0)),
            pl.BlockSpec((1, 1, _W), lambda g: (g, 0, 0)),
            pl.BlockSpec((1, 1, _W), lambda g: (n_prog + g, 0, 0)),
            pl.BlockSpec((1, 1, _W), lambda g: (2 * n_prog + g, 0, 0)),
            pl.BlockSpec((_W, E), lambda g: (g, 0)),
            pl.BlockSpec((1, E), lambda g: (0, 0)),
        ],
        out_specs=pl.BlockSpec((S, E), lambda g: (0, 0)),
        out_shape=jax.ShapeDtypeStruct((S, E), jnp.float32),
        compiler_params=pltpu.CompilerParams(
            dimension_semantics=("arbitrary",),
            vmem_limit_bytes=100 * 1024 * 1024,
        ),
    )(x, cw, cw, cw, cb, cb, cb, c_proj_w, pb)
    return out.reshape(B, S, E)


# speedup vs baseline: 6.7326x; 1.1956x over previous
import jax
import jax.numpy as jnp
from jax.experimental import pallas as pl
from jax.experimental.pallas import tpu as pltpu

_NUM_HEADS = 12
_HEAD_DIM = 64
_EMBED = 768
_GROUP = 4
_CHUNK = 512
_HPP = 2
_W = _HPP * _HEAD_DIM
_INV_SQRT = 1.0 / (_HEAD_DIM ** 0.5)

_DN_T = (((1,), (1,)), ((), ()))
_DN_N = (((1,), (0,)), ((), ()))


def _dequant_keys(k):
    a = jnp.abs(k)
    n = k.shape[1]
    w = a
    for r in (1, 2, 3):
        w = jnp.maximum(w, pltpu.roll(a, n - r, 1))
    lane = jax.lax.broadcasted_iota(jnp.int32, k.shape, 1)
    first = (lane % _GROUP) == 0
    m0 = jnp.where(first, w, 0.0)
    gm = m0
    for r in (1, 2, 3):
        gm = gm + pltpu.roll(m0, r, 1)
    scale = jnp.where(gm > 0.0, gm, 1.0)
    kq = jnp.clip(jnp.round(k / scale), -2.0, 1.0)
    return kq * scale


def _attn_kernel(x_ref, wq_ref, wk_ref, wv_ref, bq_ref, bk_ref, bv_ref,
                 wp_ref, bp_ref, out_ref):
    g = pl.program_id(0)

    @pl.when(g == 0)
    def _init():
        out_ref[...] = jnp.broadcast_to(bp_ref[...], out_ref.shape)

    x = x_ref[...]
    wq, wk, wv = wq_ref[...], wk_ref[...], wv_ref[...]
    bq, bk, bv = bq_ref[0], bk_ref[0], bv_ref[0]
    xb = x.astype(jnp.bfloat16)
    k2 = jax.lax.dot_general(x, wk, _DN_N) + bk
    v2b = (jax.lax.dot_general(xb, wv.astype(jnp.bfloat16), _DN_N,
                               preferred_element_type=jnp.float32)
           + bv).astype(jnp.bfloat16)
    kd2b = _dequant_keys(k2).astype(jnp.bfloat16)
    wqb = wq.astype(jnp.bfloat16)
    wpb = wp_ref[...].astype(jnp.bfloat16)
    seq = x.shape[0]
    for c in range(seq // _CHUNK):
        rows = pl.ds(c * _CHUNK, _CHUNK)
        xcb = x_ref[rows, :].astype(jnp.bfloat16)
        q2 = (jax.lax.dot_general(xcb, wqb, _DN_N,
                                  preferred_element_type=jnp.float32)
              + bq) * _INV_SQRT
        q2b = q2.astype(jnp.bfloat16)
        outs = []
        for hh in range(_HPP):
            lanes = slice(hh * _HEAD_DIM, (hh + 1) * _HEAD_DIM)
            s = jax.lax.dot_general(q2b[:, lanes], kd2b[:, lanes], _DN_T,
                                    preferred_element_type=jnp.float32)
            e = jnp.exp(s - jnp.max(s, axis=-1, keepdims=True))
            eb = e.astype(jnp.bfloat16)
            outs.append(jax.lax.dot_general(
                eb, v2b, (((1,), (0,)), ((), ())),
                preferred_element_type=jnp.float32)[:, lanes]
                / jnp.sum(e, axis=-1, keepdims=True))
        o2 = jnp.concatenate(outs, axis=1).astype(jnp.bfloat16)
        out_ref[rows, :] += jax.lax.dot_general(
            o2, wpb, (((1,), (0,)), ((), ())),
            preferred_element_type=jnp.float32)


def kernel(hidden_states, c_attn_w, c_attn_b, c_proj_w, c_proj_b):
    B, S, E = hidden_states.shape
    n_prog = _NUM_HEADS // _HPP
    x = hidden_states.reshape(S, E)
    cb = c_attn_b.reshape(3 * n_prog, 1, _W)
    pb = c_proj_b.reshape(1, E)
    out = pl.pallas_call(
        _attn_kernel,
        grid=(n_prog,),
        in_specs=[
            pl.BlockSpec((S, E), lambda g: (0, 0)),
            pl.BlockSpec((E, _W), lambda g: (0, g)),
            pl.BlockSpec((E, _W), lambda g: (0, n_prog + g)),
            pl.BlockSpec((E, _W), lambda g: (0, 2 * n_prog + g)),
            pl.BlockSpec((1, 1, _W), lambda g: (g, 0, 0)),
            pl.BlockSpec((1, 1, _W), lambda g: (n_prog + g, 0, 0)),
            pl.BlockSpec((1, 1, _W), lambda g: (2 * n_prog + g, 0, 0)),
            pl.BlockSpec((_W, E), lambda g: (g, 0)),
            pl.BlockSpec((1, E), lambda g: (0, 0)),
        ],
        out_specs=pl.BlockSpec((S, E), lambda g: (0, 0)),
        out_shape=jax.ShapeDtypeStruct((S, E), jnp.float32),
        compiler_params=pltpu.CompilerParams(
            dimension_semantics=("arbitrary",),
            vmem_limit_bytes=100 * 1024 * 1024,
        ),
    )(x, c_attn_w, c_attn_w, c_attn_w, cb, cb, cb, c_proj_w, pb)
    return out.reshape(B, S, E)
